# baseline (speedup 1.0000x reference)
import functools

import jax
import jax.numpy as jnp
from jax import lax
from jax.experimental import pallas as pl
from jax.experimental.pallas import tpu as pltpu
from jax.experimental.pallas import tpu_sc as plsc

NC = 2
NS = 16
CH = 128
BW = 32
NB = 4


def _pad_count(n, m):
    return ((n + m - 1) // m) * m


def _make_deg(e_pad, n_pad):
    ept = e_pad // NS
    nch = ept // CH
    sl = n_pad // NS
    mesh = plsc.VectorSubcoreMesh(core_axis_name="c", subcore_axis_name="s")

    @functools.partial(
        pl.kernel,
        out_type=[jax.ShapeDtypeStruct((n_pad,), jnp.float32)] * 2,
        mesh=mesh,
        scratch_types=[
            pltpu.VMEM((nch, 1, CH), jnp.int32),
            pltpu.VMEM((CH,), jnp.float32),
            pltpu.VMEM((sl,), jnp.float32),
            pltpu.VMEM_SHARED((n_pad,), jnp.float32),
            pltpu.SemaphoreType.DMA,
        ],
        compiler_params=pltpu.CompilerParams(use_tc_tiling_on_sc=False),
    )
    def deg(src_hbm, dst_hbm, zero_hbm, dout_hbm, din_hbm,
            idx_all, ones_b, stage_b, deg_sh, sem):
        c = lax.axis_index("c")
        s = lax.axis_index("s")
        for j in range(CH // 16):
            ones_b[pl.ds(j * 16, 16)] = jnp.ones((16,), jnp.float32)
        @pl.when(c == 0)
        def _():
            pltpu.sync_copy(src_hbm.at[pl.ds(s * nch, nch)], idx_all)

        @pl.when(c == 1)
        def _():
            pltpu.sync_copy(dst_hbm.at[pl.ds(s * nch, nch)], idx_all)

        pltpu.sync_copy(zero_hbm.at[pl.ds(s * sl, sl)], stage_b)
        pltpu.sync_copy(stage_b, deg_sh.at[pl.ds(s * sl, sl)])
        plsc.subcore_barrier()

        def fire(i, carry):
            pltpu.async_copy(ones_b, deg_sh.at[idx_all.at[i, 0]], sem, add=True)
            return carry
        lax.fori_loop(0, nch, fire, 0)

        def drain(i, carry):
            pltpu.make_async_copy(zero_hbm.at[pl.ds(0, CH)], ones_b, sem).wait()
            return carry
        lax.fori_loop(0, nch, drain, 0)

        plsc.subcore_barrier()
        pltpu.sync_copy(deg_sh.at[pl.ds(s * sl, sl)], stage_b)

        @pl.when(c == 0)
        def _():
            pltpu.sync_copy(stage_b, dout_hbm.at[pl.ds(s * sl, sl)])

        @pl.when(c == 1)
        def _():
            pltpu.sync_copy(stage_b, din_hbm.at[pl.ds(s * sl, sl)])

    return deg


def _make_agg(e_pad, n_pad, nblk):
    ept = e_pad // NS
    nch = ept // CH
    rs = n_pad // NS
    bpc = nblk // 2
    mesh = plsc.VectorSubcoreMesh(core_axis_name="c", subcore_axis_name="s")

    @functools.partial(
        pl.kernel,
        out_type=jax.ShapeDtypeStruct((n_pad, nblk * BW), jnp.float32),
        mesh=mesh,
        scratch_types=[
            pltpu.VMEM((nch, 1, CH), jnp.int32),
            pltpu.VMEM((nch, 1, CH), jnp.int32),
            [pltpu.VMEM((CH, BW), jnp.float32)] * NB,
            pltpu.VMEM((rs, BW), jnp.float32),
            pltpu.VMEM_SHARED((n_pad, BW), jnp.float32),
            [pltpu.SemaphoreType.DMA] * NB,
            [pltpu.SemaphoreType.DMA] * NB,
        ],
        compiler_params=pltpu.CompilerParams(use_tc_tiling_on_sc=False),
    )
    def agg(*args):
        h_refs = args[:nblk]
        src_hbm, dst_hbm, zero_hbm, out_hbm = args[nblk:nblk + 4]
        sidx, didx, rbufs, stage, agg_sh, sg, ss = args[nblk + 4:]
        c = lax.axis_index("c")
        s = lax.axis_index("s")
        pltpu.sync_copy(src_hbm.at[pl.ds(s * nch, nch)], sidx)
        pltpu.sync_copy(dst_hbm.at[pl.ds(s * nch, nch)], didx)

        def run_block(h_hbm, col0):
            pltpu.sync_copy(zero_hbm.at[pl.ds(s * rs, rs)], stage)
            pltpu.sync_copy(stage, agg_sh.at[pl.ds(s * rs, rs)])
            plsc.subcore_barrier()

            for j in range(NB):
                pltpu.async_copy(h_hbm.at[sidx.at[j, 0]], rbufs[j], sg[j])

            def step(g, carry):
                i0 = g * NB
                for j in range(NB):
                    pltpu.make_async_copy(
                        h_hbm.at[pl.ds(0, CH)], rbufs[j], sg[j]).wait()
                    pltpu.async_copy(
                        rbufs[j], agg_sh.at[didx.at[i0 + j, 0]], ss[j],
                        add=True)
                for j in range(NB):
                    @pl.when(i0 + NB + j < nch)
                    def _(j=j):
                        pltpu.make_async_copy(
                            h_hbm.at[pl.ds(0, CH)], rbufs[j], ss[j]).wait()
                        pltpu.async_copy(
                            h_hbm.at[sidx.at[i0 + NB + j, 0]], rbufs[j], sg[j])
                return carry

            lax.fori_loop(0, nch // NB, step, 0)
            for j in range(NB):
                pltpu.make_async_copy(
                    h_hbm.at[pl.ds(0, CH)], rbufs[j], ss[j]).wait()

            plsc.subcore_barrier()
            pltpu.sync_copy(agg_sh.at[pl.ds(s * rs, rs)], stage)
            pltpu.sync_copy(
                stage, out_hbm.at[pl.ds(s * rs, rs), pl.ds(col0, BW)])
            plsc.subcore_barrier()

        for q in range(bpc):
            @pl.when(c == 0)
            def _(q=q):
                run_block(h_refs[q], q * BW)

            @pl.when(c == 1)
            def _(q=q):
                run_block(h_refs[bpc + q], (bpc + q) * BW)

    return agg


def _make_agg_split(e_pad, n_pad, dw):
    ept = e_pad // (NC * NS)
    nch = ept // CH
    rs = n_pad // NS
    mesh = plsc.VectorSubcoreMesh(core_axis_name="c", subcore_axis_name="s")

    @functools.partial(
        pl.kernel,
        out_type=[jax.ShapeDtypeStruct((n_pad, dw), jnp.float32)] * 2,
        mesh=mesh,
        scratch_types=[
            pltpu.VMEM((nch, 1, CH), jnp.int32),
            pltpu.VMEM((nch, 1, CH), jnp.int32),
            [pltpu.VMEM((CH, dw), jnp.float32)] * NB,
            pltpu.VMEM((rs, dw), jnp.float32),
            pltpu.VMEM_SHARED((n_pad, dw), jnp.float32),
            [pltpu.SemaphoreType.DMA] * NB,
            [pltpu.SemaphoreType.DMA] * NB,
        ],
        compiler_params=pltpu.CompilerParams(use_tc_tiling_on_sc=False),
    )
    def agg(h_hbm, src_hbm, dst_hbm, zero_hbm, pa_hbm, pb_hbm,
            sidx, didx, rbufs, stage, agg_sh, sg, ss):
        c = lax.axis_index("c")
        s = lax.axis_index("s")
        wid = c * NS + s
        pltpu.sync_copy(src_hbm.at[pl.ds(wid * nch, nch)], sidx)
        pltpu.sync_copy(dst_hbm.at[pl.ds(wid * nch, nch)], didx)
        pltpu.sync_copy(zero_hbm.at[pl.ds(s * rs, rs)], stage)
        pltpu.sync_copy(stage, agg_sh.at[pl.ds(s * rs, rs)])
        plsc.subcore_barrier()

        for j in range(NB):
            pltpu.async_copy(h_hbm.at[sidx.at[j, 0]], rbufs[j], sg[j])

        def step(g, carry):
            i0 = g * NB
            for j in range(NB):
                pltpu.make_async_copy(
                    zero_hbm.at[pl.ds(0, CH)], rbufs[j], sg[j]).wait()
                pltpu.async_copy(
                    rbufs[j], agg_sh.at[didx.at[i0 + j, 0]], ss[j], add=True)
            for j in range(NB):
                @pl.when(i0 + NB + j < nch)
                def _(j=j):
                    pltpu.make_async_copy(
                        zero_hbm.at[pl.ds(0, CH)], rbufs[j], ss[j]).wait()
                    pltpu.async_copy(
                        h_hbm.at[sidx.at[i0 + NB + j, 0]], rbufs[j], sg[j])
            return carry

        lax.fori_loop(0, nch // NB, step, 0)
        for j in range(NB):
            pltpu.make_async_copy(
                zero_hbm.at[pl.ds(0, CH)], rbufs[j], ss[j]).wait()

        plsc.subcore_barrier()
        pltpu.sync_copy(agg_sh.at[pl.ds(s * rs, rs)], stage)

        @pl.when(c == 0)
        def _():
            pltpu.sync_copy(stage, pa_hbm.at[pl.ds(s * rs, rs)])

        @pl.when(c == 1)
        def _():
            pltpu.sync_copy(stage, pb_hbm.at[pl.ds(s * rs, rs)])

    return agg


def _mm0a_body(feat_ref, w0_ref, y_ref):
    y_ref[...] = jnp.dot(feat_ref[...], w0_ref[...],
                         preferred_element_type=jnp.float32,
                         precision=lax.Precision.HIGHEST)


def _mm0b_body(y_ref, do_ref, di_ref,
               h0_ref, h1_ref, h2_ref, h3_ref, ns_ref, nd_ref):
    ns = lax.rsqrt(jnp.maximum(do_ref[...], 1.0))
    nd = lax.rsqrt(jnp.maximum(di_ref[...], 1.0))
    ns_ref[...] = ns
    nd_ref[...] = nd
    h = y_ref[...] * ns
    for q, href in enumerate([h0_ref, h1_ref, h2_ref, h3_ref]):
        href[...] = h[:, q * BW:(q + 1) * BW]


def _mm1_body(p_ref, nd_ref, ns_ref, b0_ref, w1_ref, h_ref):
    t = jnp.maximum(p_ref[...] * nd_ref[...] + b0_ref[...], 0.0) * ns_ref[...]
    h_ref[...] = jnp.dot(t, w1_ref[...], preferred_element_type=jnp.float32,
                         precision=lax.Precision.HIGHEST)


def _make_fin_body(ncls):
    def _fin_body(qa_ref, qb_ref, nd_ref, b1_ref, o_ref):
        q = qa_ref[...] + qb_ref[...]
        o_ref[...] = q[:, :ncls] * nd_ref[...] + b1_ref[...]
    return _fin_body


def kernel(feat, edge_index, W0, b0, W1, b1):
    n, d_in = feat.shape
    d_hid = W0.shape[1]
    n_cls = W1.shape[1]
    e = edge_index.shape[1]
    dw1 = _pad_count(n_cls, 16)

    n_pad = _pad_count(n + 1, NS * 8)
    e_pad = _pad_count(e, NS * CH * 8)
    rb = n_pad // NS
    grid = (n_pad // rb,)

    src = edge_index[0].astype(jnp.int32)
    dst = edge_index[1].astype(jnp.int32)
    ipad = jnp.full((e_pad - e,), n, jnp.int32)
    src_p = jnp.concatenate([src, ipad]).reshape(-1, 1, CH)
    dst_p = jnp.concatenate([dst, ipad]).reshape(-1, 1, CH)
    feat_p = jnp.concatenate(
        [feat, jnp.zeros((n_pad - n, d_in), jnp.float32)])

    zero1 = jnp.zeros((n_pad,), jnp.float32)
    zero_b = jnp.zeros((n_pad, BW), jnp.float32)
    zero_w = jnp.zeros((n_pad, dw1), jnp.float32)
    W1p = jnp.zeros((d_hid, dw1), jnp.float32).at[:, :n_cls].set(W1)

    deg_out, deg_in = _make_deg(e_pad, n_pad)(src_p, dst_p, zero1)
    do2 = deg_out.reshape(n_pad, 1)
    di2 = deg_in.reshape(n_pad, 1)

    y0 = pl.pallas_call(
        _mm0a_body,
        grid=grid,
        in_specs=[
            pl.BlockSpec((rb, d_in), lambda i: (i, 0)),
            pl.BlockSpec((d_in, d_hid), lambda i: (0, 0)),
        ],
        out_specs=pl.BlockSpec((rb, d_hid), lambda i: (i, 0)),
        out_shape=jax.ShapeDtypeStruct((n_pad, d_hid), jnp.float32),
    )(feat_p, W0)

    h00, h01, h02, h03, nsrc, ndst = pl.pallas_call(
        _mm0b_body,
        grid=grid,
        in_specs=[
            pl.BlockSpec((rb, d_hid), lambda i: (i, 0)),
            pl.BlockSpec((rb, 1), lambda i: (i, 0)),
            pl.BlockSpec((rb, 1), lambda i: (i, 0)),
        ],
        out_specs=[pl.BlockSpec((rb, BW), lambda i: (i, 0))] * 4
        + [pl.BlockSpec((rb, 1), lambda i: (i, 0))] * 2,
        out_shape=[jax.ShapeDtypeStruct((n_pad, BW), jnp.float32)] * 4
        + [jax.ShapeDtypeStruct((n_pad, 1), jnp.float32)] * 2,
    )(y0, do2, di2)

    p = _make_agg(e_pad, n_pad, 4)(h00, h01, h02, h03, src_p, dst_p, zero_b)

    h1 = pl.pallas_call(
        _mm1_body,
        grid=grid,
        in_specs=[
            pl.BlockSpec((rb, d_hid), lambda i: (i, 0)),
            pl.BlockSpec((rb, 1), lambda i: (i, 0)),
            pl.BlockSpec((rb, 1), lambda i: (i, 0)),
            pl.BlockSpec((1, d_hid), lambda i: (0, 0)),
            pl.BlockSpec((d_hid, dw1), lambda i: (0, 0)),
        ],
        out_specs=pl.BlockSpec((rb, dw1), lambda i: (i, 0)),
        out_shape=jax.ShapeDtypeStruct((n_pad, dw1), jnp.float32),
    )(p, ndst, nsrc, b0.reshape(1, d_hid), W1p)

    qa, qb = _make_agg_split(e_pad, n_pad, dw1)(h1, src_p, dst_p, zero_w)

    out = pl.pallas_call(
        _make_fin_body(n_cls),
        grid=grid,
        in_specs=[
            pl.BlockSpec((rb, dw1), lambda i: (i, 0)),
            pl.BlockSpec((rb, dw1), lambda i: (i, 0)),
            pl.BlockSpec((rb, 1), lambda i: (i, 0)),
            pl.BlockSpec((1, n_cls), lambda i: (0, 0)),
        ],
        out_specs=pl.BlockSpec((rb, n_cls), lambda i: (i, 0)),
        out_shape=jax.ShapeDtypeStruct((n, n_cls), jnp.float32),
    )(qa, qb, ndst, b1.reshape(1, n_cls))

    return out


# speedup vs baseline: 1.9358x; 1.9358x over previous
import functools

import jax
import jax.numpy as jnp
from jax import lax
from jax.experimental import pallas as pl
from jax.experimental.pallas import tpu as pltpu
from jax.experimental.pallas import tpu_sc as plsc

NC = 2
NS = 16
CH = 128
BW = 32
NB = 4


def _pad_count(n, m):
    return ((n + m - 1) // m) * m


def _make_deg(e_pad, n_pad):
    ept = e_pad // NS
    nch = ept // CH
    sl = n_pad // NS
    mesh = plsc.VectorSubcoreMesh(core_axis_name="c", subcore_axis_name="s")

    @functools.partial(
        pl.kernel,
        out_type=[jax.ShapeDtypeStruct((n_pad,), jnp.float32)] * 2,
        mesh=mesh,
        scratch_types=[
            pltpu.VMEM((nch, 1, CH), jnp.int32),
            pltpu.VMEM((CH,), jnp.float32),
            pltpu.VMEM((sl,), jnp.float32),
            pltpu.VMEM_SHARED((n_pad,), jnp.float32),
            pltpu.SemaphoreType.DMA,
        ],
        compiler_params=pltpu.CompilerParams(use_tc_tiling_on_sc=False),
    )
    def deg(src_hbm, dst_hbm, zero_hbm, dout_hbm, din_hbm,
            idx_all, ones_b, stage_b, deg_sh, sem):
        c = lax.axis_index("c")
        s = lax.axis_index("s")
        for j in range(CH // 16):
            ones_b[pl.ds(j * 16, 16)] = jnp.ones((16,), jnp.float32)
        @pl.when(c == 0)
        def _():
            pltpu.sync_copy(src_hbm.at[pl.ds(s * nch, nch)], idx_all)

        @pl.when(c == 1)
        def _():
            pltpu.sync_copy(dst_hbm.at[pl.ds(s * nch, nch)], idx_all)

        pltpu.sync_copy(zero_hbm.at[pl.ds(s * sl, sl)], stage_b)
        pltpu.sync_copy(stage_b, deg_sh.at[pl.ds(s * sl, sl)])
        plsc.subcore_barrier()

        def fire(i, carry):
            pltpu.async_copy(ones_b, deg_sh.at[idx_all.at[i, 0]], sem, add=True)
            return carry
        lax.fori_loop(0, nch, fire, 0)

        def drain(i, carry):
            pltpu.make_async_copy(zero_hbm.at[pl.ds(0, CH)], ones_b, sem).wait()
            return carry
        lax.fori_loop(0, nch, drain, 0)

        plsc.subcore_barrier()
        pltpu.sync_copy(deg_sh.at[pl.ds(s * sl, sl)], stage_b)

        @pl.when(c == 0)
        def _():
            pltpu.sync_copy(stage_b, dout_hbm.at[pl.ds(s * sl, sl)])

        @pl.when(c == 1)
        def _():
            pltpu.sync_copy(stage_b, din_hbm.at[pl.ds(s * sl, sl)])

    return deg


def _make_agg(e_pad, n_pad, nblk):
    ept = e_pad // NS
    nch = ept // CH
    rs = n_pad // NS
    bpc = nblk // 2
    mesh = plsc.VectorSubcoreMesh(core_axis_name="c", subcore_axis_name="s")

    @functools.partial(
        pl.kernel,
        out_type=jax.ShapeDtypeStruct((n_pad, nblk * BW), jnp.float32),
        mesh=mesh,
        scratch_types=[
            pltpu.VMEM((nch, 1, CH), jnp.int32),
            pltpu.VMEM((nch, 1, CH), jnp.int32),
            [pltpu.VMEM((CH, BW), jnp.float32)] * NB,
            pltpu.VMEM((rs, BW), jnp.float32),
            pltpu.VMEM_SHARED((n_pad, BW), jnp.float32),
            [pltpu.SemaphoreType.DMA] * NB,
            [pltpu.SemaphoreType.DMA] * NB,
        ],
        compiler_params=pltpu.CompilerParams(use_tc_tiling_on_sc=False),
    )
    def agg(*args):
        h_refs = args[:nblk]
        src_hbm, dst_hbm, zero_hbm, out_hbm = args[nblk:nblk + 4]
        sidx, didx, rbufs, stage, agg_sh, sg, ss = args[nblk + 4:]
        c = lax.axis_index("c")
        s = lax.axis_index("s")
        pltpu.sync_copy(src_hbm.at[pl.ds(s * nch, nch)], sidx)
        pltpu.sync_copy(dst_hbm.at[pl.ds(s * nch, nch)], didx)

        def run_block(h_hbm, col0):
            pltpu.sync_copy(zero_hbm.at[pl.ds(s * rs, rs)], stage)
            pltpu.sync_copy(stage, agg_sh.at[pl.ds(s * rs, rs)])
            plsc.subcore_barrier()

            for j in range(NB):
                pltpu.async_copy(h_hbm.at[sidx.at[j, 0]], rbufs[j], sg[j])

            def step(g, carry):
                i0 = g * NB
                for j in range(NB):
                    pltpu.make_async_copy(
                        h_hbm.at[pl.ds(0, CH)], rbufs[j], sg[j]).wait()
                    pltpu.async_copy(
                        rbufs[j], agg_sh.at[didx.at[i0 + j, 0]], ss[j],
                        add=True)
                for j in range(NB):
                    @pl.when(i0 + NB + j < nch)
                    def _(j=j):
                        pltpu.make_async_copy(
                            h_hbm.at[pl.ds(0, CH)], rbufs[j], ss[j]).wait()
                        pltpu.async_copy(
                            h_hbm.at[sidx.at[i0 + NB + j, 0]], rbufs[j], sg[j])
                return carry

            lax.fori_loop(0, nch // NB, step, 0)
            for j in range(NB):
                pltpu.make_async_copy(
                    h_hbm.at[pl.ds(0, CH)], rbufs[j], ss[j]).wait()

            plsc.subcore_barrier()
            pltpu.sync_copy(agg_sh.at[pl.ds(s * rs, rs)], stage)
            pltpu.sync_copy(
                stage, out_hbm.at[pl.ds(s * rs, rs), pl.ds(col0, BW)])
            plsc.subcore_barrier()

        for q in range(bpc):
            @pl.when(c == 0)
            def _(q=q):
                run_block(h_refs[q], q * BW)

            @pl.when(c == 1)
            def _(q=q):
                run_block(h_refs[bpc + q], (bpc + q) * BW)

    return agg


def _make_agg_split(e_pad, n_pad, dw):
    ept = e_pad // (NC * NS)
    nch = ept // CH
    rs = n_pad // NS
    mesh = plsc.VectorSubcoreMesh(core_axis_name="c", subcore_axis_name="s")

    @functools.partial(
        pl.kernel,
        out_type=[jax.ShapeDtypeStruct((n_pad, dw), jnp.float32)] * 2,
        mesh=mesh,
        scratch_types=[
            pltpu.VMEM((nch, 1, CH), jnp.int32),
            pltpu.VMEM((nch, 1, CH), jnp.int32),
            [pltpu.VMEM((CH, dw), jnp.float32)] * NB,
            pltpu.VMEM((rs, dw), jnp.float32),
            pltpu.VMEM_SHARED((n_pad, dw), jnp.float32),
            [pltpu.SemaphoreType.DMA] * NB,
            [pltpu.SemaphoreType.DMA] * NB,
        ],
        compiler_params=pltpu.CompilerParams(use_tc_tiling_on_sc=False),
    )
    def agg(h_hbm, src_hbm, dst_hbm, zero_hbm, pa_hbm, pb_hbm,
            sidx, didx, rbufs, stage, agg_sh, sg, ss):
        c = lax.axis_index("c")
        s = lax.axis_index("s")
        wid = c * NS + s
        pltpu.sync_copy(src_hbm.at[pl.ds(wid * nch, nch)], sidx)
        pltpu.sync_copy(dst_hbm.at[pl.ds(wid * nch, nch)], didx)
        pltpu.sync_copy(zero_hbm.at[pl.ds(s * rs, rs)], stage)
        pltpu.sync_copy(stage, agg_sh.at[pl.ds(s * rs, rs)])
        plsc.subcore_barrier()

        for j in range(NB):
            pltpu.async_copy(h_hbm.at[sidx.at[j, 0]], rbufs[j], sg[j])

        def step(g, carry):
            i0 = g * NB
            for j in range(NB):
                pltpu.make_async_copy(
                    zero_hbm.at[pl.ds(0, CH)], rbufs[j], sg[j]).wait()
                pltpu.async_copy(
                    rbufs[j], agg_sh.at[didx.at[i0 + j, 0]], ss[j], add=True)
            for j in range(NB):
                @pl.when(i0 + NB + j < nch)
                def _(j=j):
                    pltpu.make_async_copy(
                        zero_hbm.at[pl.ds(0, CH)], rbufs[j], ss[j]).wait()
                    pltpu.async_copy(
                        h_hbm.at[sidx.at[i0 + NB + j, 0]], rbufs[j], sg[j])
            return carry

        lax.fori_loop(0, nch // NB, step, 0)
        for j in range(NB):
            pltpu.make_async_copy(
                zero_hbm.at[pl.ds(0, CH)], rbufs[j], ss[j]).wait()

        plsc.subcore_barrier()
        pltpu.sync_copy(agg_sh.at[pl.ds(s * rs, rs)], stage)

        @pl.when(c == 0)
        def _():
            pltpu.sync_copy(stage, pa_hbm.at[pl.ds(s * rs, rs)])

        @pl.when(c == 1)
        def _():
            pltpu.sync_copy(stage, pb_hbm.at[pl.ds(s * rs, rs)])

    return agg


def _mm0a_body(feat_ref, w0_ref, y_ref):
    y_ref[...] = jnp.dot(feat_ref[...], w0_ref[...],
                         preferred_element_type=jnp.float32,
                         precision=lax.Precision.HIGHEST)


def _mm0b_body(y_ref, do_ref, di_ref,
               h0_ref, h1_ref, h2_ref, h3_ref, ns_ref, nd_ref):
    ns = lax.rsqrt(jnp.maximum(do_ref[...], 1.0))
    nd = lax.rsqrt(jnp.maximum(di_ref[...], 1.0))
    ns_ref[...] = ns
    nd_ref[...] = nd
    h = y_ref[...] * ns
    for q, href in enumerate([h0_ref, h1_ref, h2_ref, h3_ref]):
        href[...] = h[:, q * BW:(q + 1) * BW]


def _mm1_body(p_ref, nd_ref, ns_ref, b0_ref, w1_ref, h_ref):
    t = jnp.maximum(p_ref[...] * nd_ref[...] + b0_ref[...], 0.0) * ns_ref[...]
    h_ref[...] = jnp.dot(t, w1_ref[...], preferred_element_type=jnp.float32,
                         precision=lax.Precision.HIGHEST)


def _make_fin_body(ncls):
    def _fin_body(qa_ref, qb_ref, nd_ref, b1_ref, o_ref):
        q = qa_ref[...] + qb_ref[...]
        o_ref[...] = q[:, :ncls] * nd_ref[...] + b1_ref[...]
    return _fin_body


def kernel(feat, edge_index, W0, b0, W1, b1):
    n, d_in = feat.shape
    d_hid = W0.shape[1]
    n_cls = W1.shape[1]
    e = edge_index.shape[1]
    dw1 = _pad_count(n_cls, 16)

    n_pad = _pad_count(n + 1, NS * 8)
    e_pad = _pad_count(e, NS * CH * 8)
    rb = n_pad // NS
    grid = (n_pad // rb,)

    src = edge_index[0].astype(jnp.int32)
    dst = edge_index[1].astype(jnp.int32)
    ipad = n + (jnp.arange(e_pad - e, dtype=jnp.int32) % (n_pad - n))
    src_p = jnp.concatenate([src, ipad]).reshape(-1, 1, CH)
    dst_p = jnp.concatenate([dst, ipad]).reshape(-1, 1, CH)
    feat_p = jnp.concatenate(
        [feat, jnp.zeros((n_pad - n, d_in), jnp.float32)])

    zero1 = jnp.zeros((n_pad,), jnp.float32)
    zero_b = jnp.zeros((n_pad, BW), jnp.float32)
    zero_w = jnp.zeros((n_pad, dw1), jnp.float32)
    W1p = jnp.zeros((d_hid, dw1), jnp.float32).at[:, :n_cls].set(W1)

    deg_out, deg_in = _make_deg(e_pad, n_pad)(src_p, dst_p, zero1)
    do2 = deg_out.reshape(n_pad, 1)
    di2 = deg_in.reshape(n_pad, 1)

    y0 = pl.pallas_call(
        _mm0a_body,
        grid=grid,
        in_specs=[
            pl.BlockSpec((rb, d_in), lambda i: (i, 0)),
            pl.BlockSpec((d_in, d_hid), lambda i: (0, 0)),
        ],
        out_specs=pl.BlockSpec((rb, d_hid), lambda i: (i, 0)),
        out_shape=jax.ShapeDtypeStruct((n_pad, d_hid), jnp.float32),
    )(feat_p, W0)

    h00, h01, h02, h03, nsrc, ndst = pl.pallas_call(
        _mm0b_body,
        grid=grid,
        in_specs=[
            pl.BlockSpec((rb, d_hid), lambda i: (i, 0)),
            pl.BlockSpec((rb, 1), lambda i: (i, 0)),
            pl.BlockSpec((rb, 1), lambda i: (i, 0)),
        ],
        out_specs=[pl.BlockSpec((rb, BW), lambda i: (i, 0))] * 4
        + [pl.BlockSpec((rb, 1), lambda i: (i, 0))] * 2,
        out_shape=[jax.ShapeDtypeStruct((n_pad, BW), jnp.float32)] * 4
        + [jax.ShapeDtypeStruct((n_pad, 1), jnp.float32)] * 2,
    )(y0, do2, di2)

    p = _make_agg(e_pad, n_pad, 4)(h00, h01, h02, h03, src_p, dst_p, zero_b)

    h1 = pl.pallas_call(
        _mm1_body,
        grid=grid,
        in_specs=[
            pl.BlockSpec((rb, d_hid), lambda i: (i, 0)),
            pl.BlockSpec((rb, 1), lambda i: (i, 0)),
            pl.BlockSpec((rb, 1), lambda i: (i, 0)),
            pl.BlockSpec((1, d_hid), lambda i: (0, 0)),
            pl.BlockSpec((d_hid, dw1), lambda i: (0, 0)),
        ],
        out_specs=pl.BlockSpec((rb, dw1), lambda i: (i, 0)),
        out_shape=jax.ShapeDtypeStruct((n_pad, dw1), jnp.float32),
    )(p, ndst, nsrc, b0.reshape(1, d_hid), W1p)

    qa, qb = _make_agg_split(e_pad, n_pad, dw1)(h1, src_p, dst_p, zero_w)

    out = pl.pallas_call(
        _make_fin_body(n_cls),
        grid=grid,
        in_specs=[
            pl.BlockSpec((rb, dw1), lambda i: (i, 0)),
            pl.BlockSpec((rb, dw1), lambda i: (i, 0)),
            pl.BlockSpec((rb, 1), lambda i: (i, 0)),
            pl.BlockSpec((1, n_cls), lambda i: (0, 0)),
        ],
        out_specs=pl.BlockSpec((rb, n_cls), lambda i: (i, 0)),
        out_shape=jax.ShapeDtypeStruct((n, n_cls), jnp.float32),
    )(qa, qb, ndst, b1.reshape(1, n_cls))

    return out
